# Initial kernel scaffold; baseline (speedup 1.0000x reference)
import jax
import jax.numpy as jnp
from jax.experimental import pallas as pl


def kernel(x_user, x_item, edge_follows, edge_clicks, edge_clicked_by, W_follows, b_follows, W_clicks, b_clicks, W_clicked_by, b_clicked_by):
    raise NotImplementedError("write your pallas kernel here")


# speedup vs baseline: 3.8028x; 3.8028x over previous
import functools

import jax
import jax.numpy as jnp
from jax import lax
from jax.experimental import pallas as pl
from jax.experimental.pallas import tpu as pltpu
from jax.experimental.pallas import tpu_sc as plsc

N_NODE = 10000
E = 320000
D = 128
DP = 144
NROWS = 10112
JUNK = 10000
NC, NS = 2, 16
RPT = NROWS // NS
K = 128
C1 = 158
C2 = 80
BM = 1000


def _mm_body(xu, xi, wf, wc, wcb, bf, bc, bcb, of, oc, ocb):
    colpad = (lax.broadcasted_iota(jnp.int32, (BM, DP - D), 1) == 0).astype(
        jnp.float32)
    of[:, :D] = jnp.dot(xu[...], wf[...], preferred_element_type=jnp.float32) + bf[...]
    of[:, D:] = colpad
    oc[:, :D] = jnp.dot(xu[...], wc[...], preferred_element_type=jnp.float32) + bc[...]
    oc[:, D:] = colpad
    ocb[:, :D] = jnp.dot(xi[...], wcb[...], preferred_element_type=jnp.float32) + bcb[...]
    ocb[:, D:] = colpad


def _make_tables(x_user, x_item, W_f, b_f, W_c, b_c, W_cb, b_cb):
    grid = N_NODE // BM
    full_w = pl.BlockSpec((D, D), lambda i: (0, 0))
    full_b = pl.BlockSpec((1, D), lambda i: (0, 0))
    row_blk = pl.BlockSpec((BM, D), lambda i: (i, 0))
    out_blk = pl.BlockSpec((BM, DP), lambda i: (i, 0))
    out_sds = jax.ShapeDtypeStruct((N_NODE, DP), jnp.float32)
    return pl.pallas_call(
        _mm_body,
        grid=(grid,),
        in_specs=[row_blk, row_blk, full_w, full_w, full_w,
                  full_b, full_b, full_b],
        out_specs=(out_blk, out_blk, out_blk),
        out_shape=(out_sds, out_sds, out_sds),
    )(x_user, x_item, W_f, W_c, W_cb, b_f.reshape(1, D), b_c.reshape(1, D),
      b_cb.reshape(1, D))


def _sc_body(whp_f, whp_c, whp_cb, s1, d1, s2, d2, zeros_hbm,
             out1, out2, idx_s, idx_d, rows, acc, sem):
    cc = lax.axis_index("c")
    tid = lax.axis_index("s")

    def run_phase(table, s_e, d_e, nchunks, out):
        pltpu.sync_copy(zeros_hbm, acc.at[pl.ds(tid * RPT, RPT)])
        plsc.subcore_barrier()

        def body(j, carry):
            pltpu.sync_copy(s_e.at[cc, tid, j], idx_s)
            pltpu.sync_copy(d_e.at[cc, tid, j], idx_d)
            pltpu.async_copy(table.at[idx_s], rows, sem).wait()
            pltpu.sync_copy(rows, acc.at[idx_d], add=True)
            return carry

        lax.fori_loop(0, nchunks, body, 0)
        plsc.subcore_barrier()
        pltpu.sync_copy(acc.at[pl.ds(tid * RPT, RPT)],
                        out.at[cc, pl.ds(tid * RPT, RPT)])

    @pl.when(cc == 0)
    def _():
        run_phase(whp_f, s1, d1, C1, out1)

    @pl.when(cc == 1)
    def _():
        run_phase(whp_cb, s1, d1, C1, out1)

    run_phase(whp_c, s2, d2, C2, out2)


def _comb_body(o1, o2, hu, hi):
    sf = o1[0, :, :D]
    cf = o1[0, :, D:D + 1]
    scb = o1[1, :, :D]
    ccb = o1[1, :, D:D + 1]
    hu[...] = sf / jnp.maximum(cf, 1.0) + scb / jnp.maximum(ccb, 1.0)
    s0 = o2[0, :, :D]
    c0 = o2[0, :, D:D + 1]
    s1_ = o2[1, :, :D]
    c1 = o2[1, :, D:D + 1]
    hi[...] = (s0 + s1_) / jnp.maximum(c0 + c1, 1.0)


def _pad_edges(idx, fill, per_tile_chunks):
    total = NS * per_tile_chunks * K
    out = jnp.full((total,), fill, dtype=jnp.int32).at[: idx.shape[0]].set(idx)
    return out.reshape(NS, per_tile_chunks, K)


def kernel(x_user, x_item, edge_follows, edge_clicks, edge_clicked_by,
           W_follows, b_follows, W_clicks, b_clicks, W_clicked_by,
           b_clicked_by):
    whp_f, whp_c, whp_cb = _make_tables(
        x_user, x_item, W_follows, b_follows, W_clicks, b_clicks,
        W_clicked_by, b_clicked_by)

    s1 = jnp.stack([_pad_edges(edge_follows[0], 0, C1),
                    _pad_edges(edge_clicked_by[0], 0, C1)])
    d1 = jnp.stack([_pad_edges(edge_follows[1], JUNK, C1),
                    _pad_edges(edge_clicked_by[1], JUNK, C1)])
    half = E // 2
    s2 = jnp.stack([_pad_edges(edge_clicks[0, :half], 0, C2),
                    _pad_edges(edge_clicks[0, half:], 0, C2)])
    d2 = jnp.stack([_pad_edges(edge_clicks[1, :half], JUNK, C2),
                    _pad_edges(edge_clicks[1, half:], JUNK, C2)])
    zeros_hbm = jnp.zeros((RPT, DP), jnp.float32)

    sc = pl.kernel(
        _sc_body,
        out_type=(jax.ShapeDtypeStruct((NC, NROWS, DP), jnp.float32),
                  jax.ShapeDtypeStruct((NC, NROWS, DP), jnp.float32)),
        mesh=plsc.VectorSubcoreMesh(core_axis_name="c", subcore_axis_name="s"),
        scratch_types=[
            pltpu.VMEM((K,), jnp.int32),
            pltpu.VMEM((K,), jnp.int32),
            pltpu.VMEM((K, DP), jnp.float32),
            pltpu.VMEM_SHARED((NROWS, DP), jnp.float32),
            pltpu.SemaphoreType.DMA,
        ],
        compiler_params=pltpu.CompilerParams(use_tc_tiling_on_sc=False),
    )
    out1, out2 = sc(whp_f, whp_c, whp_cb, s1, d1, s2, d2, zeros_hbm)

    grid = N_NODE // BM
    h_user, h_item = pl.pallas_call(
        _comb_body,
        grid=(grid,),
        in_specs=[pl.BlockSpec((NC, BM, DP), lambda i: (0, i, 0)),
                  pl.BlockSpec((NC, BM, DP), lambda i: (0, i, 0))],
        out_specs=(pl.BlockSpec((BM, D), lambda i: (i, 0)),
                   pl.BlockSpec((BM, D), lambda i: (i, 0))),
        out_shape=(jax.ShapeDtypeStruct((N_NODE, D), jnp.float32),
                   jax.ShapeDtypeStruct((N_NODE, D), jnp.float32)),
    )(out1, out2)
    return (h_user, h_item)


# speedup vs baseline: 4.7651x; 1.2530x over previous
import functools

import jax
import jax.numpy as jnp
from jax import lax
from jax.experimental import pallas as pl
from jax.experimental.pallas import tpu as pltpu
from jax.experimental.pallas import tpu_sc as plsc

N_NODE = 10000
E = 320000
D = 128
DP = 144
NROWS = 10112
JUNK = 10000
NC, NS = 2, 16
RPT = NROWS // NS
K = 128
C1 = 158
C2 = 80
BM = 1000


def _mm_body(xu, xi, wf, wc, wcb, bf, bc, bcb, of, oc, ocb):
    colpad = (lax.broadcasted_iota(jnp.int32, (BM, DP - D), 1) == 0).astype(
        jnp.float32)
    of[:, :D] = jnp.dot(xu[...], wf[...], preferred_element_type=jnp.float32) + bf[...]
    of[:, D:] = colpad
    oc[:, :D] = jnp.dot(xu[...], wc[...], preferred_element_type=jnp.float32) + bc[...]
    oc[:, D:] = colpad
    ocb[:, :D] = jnp.dot(xi[...], wcb[...], preferred_element_type=jnp.float32) + bcb[...]
    ocb[:, D:] = colpad


def _make_tables(x_user, x_item, W_f, b_f, W_c, b_c, W_cb, b_cb):
    grid = N_NODE // BM
    full_w = pl.BlockSpec((D, D), lambda i: (0, 0))
    full_b = pl.BlockSpec((1, D), lambda i: (0, 0))
    row_blk = pl.BlockSpec((BM, D), lambda i: (i, 0))
    out_blk = pl.BlockSpec((BM, DP), lambda i: (i, 0))
    out_sds = jax.ShapeDtypeStruct((N_NODE, DP), jnp.float32)
    return pl.pallas_call(
        _mm_body,
        grid=(grid,),
        in_specs=[row_blk, row_blk, full_w, full_w, full_w,
                  full_b, full_b, full_b],
        out_specs=(out_blk, out_blk, out_blk),
        out_shape=(out_sds, out_sds, out_sds),
    )(x_user, x_item, W_f, W_c, W_cb, b_f.reshape(1, D), b_c.reshape(1, D),
      b_cb.reshape(1, D))


def _sc_body(whp_f, whp_c, whp_cb, s1, d1, s2, d2, zeros_hbm,
             out1, out2, s_all, d_all, rows, acc, semg0, semg1):
    cc = lax.axis_index("c")
    tid = lax.axis_index("s")
    sems = (semg0, semg1)

    def run_phase(table, s_e, d_e, nchunks, out):
        pltpu.sync_copy(zeros_hbm, acc.at[pl.ds(tid * RPT, RPT)])
        plsc.subcore_barrier()

        def start_gather(b, j):
            pltpu.sync_copy(s_e.at[cc, tid, j], s_all.at[b])
            pltpu.sync_copy(d_e.at[cc, tid, j], d_all.at[b])
            pltpu.async_copy(table.at[s_all.at[b]], rows.at[b], sems[b])

        def wait_gather(b):
            pltpu.make_async_copy(table.at[s_all.at[b]], rows.at[b],
                                  sems[b]).wait()

        def scatter(b):
            pltpu.sync_copy(rows.at[b], acc.at[d_all.at[b]], add=True)

        start_gather(0, 0)

        def body2(jj, carry):
            j0 = jj * 2
            j1 = j0 + 1
            start_gather(1, j1)
            wait_gather(0)
            scatter(0)

            @pl.when(j0 + 2 < nchunks)
            def _():
                start_gather(0, j0 + 2)

            wait_gather(1)
            scatter(1)
            return carry

        lax.fori_loop(0, nchunks // 2, body2, 0)
        plsc.subcore_barrier()
        pltpu.sync_copy(acc.at[pl.ds(tid * RPT, RPT)],
                        out.at[cc, pl.ds(tid * RPT, RPT)])

    @pl.when(cc == 0)
    def _():
        run_phase(whp_f, s1, d1, C1, out1)

    @pl.when(cc == 1)
    def _():
        run_phase(whp_cb, s1, d1, C1, out1)

    run_phase(whp_c, s2, d2, C2, out2)


def _comb_body(o1, o2, hu, hi):
    sf = o1[0, :, :D]
    cf = o1[0, :, D:D + 1]
    scb = o1[1, :, :D]
    ccb = o1[1, :, D:D + 1]
    hu[...] = sf / jnp.maximum(cf, 1.0) + scb / jnp.maximum(ccb, 1.0)
    s0 = o2[0, :, :D]
    c0 = o2[0, :, D:D + 1]
    s1_ = o2[1, :, :D]
    c1 = o2[1, :, D:D + 1]
    hi[...] = (s0 + s1_) / jnp.maximum(c0 + c1, 1.0)


def _pad_edges(idx, fill, per_tile_chunks):
    total = NS * per_tile_chunks * K
    out = jnp.full((total,), fill, dtype=jnp.int32).at[: idx.shape[0]].set(idx)
    return out.reshape(NS, per_tile_chunks, K)


def kernel(x_user, x_item, edge_follows, edge_clicks, edge_clicked_by,
           W_follows, b_follows, W_clicks, b_clicks, W_clicked_by,
           b_clicked_by):
    whp_f, whp_c, whp_cb = _make_tables(
        x_user, x_item, W_follows, b_follows, W_clicks, b_clicks,
        W_clicked_by, b_clicked_by)

    s1 = jnp.stack([_pad_edges(edge_follows[0], 0, C1),
                    _pad_edges(edge_clicked_by[0], 0, C1)])
    d1 = jnp.stack([_pad_edges(edge_follows[1], JUNK, C1),
                    _pad_edges(edge_clicked_by[1], JUNK, C1)])
    half = E // 2
    s2 = jnp.stack([_pad_edges(edge_clicks[0, :half], 0, C2),
                    _pad_edges(edge_clicks[0, half:], 0, C2)])
    d2 = jnp.stack([_pad_edges(edge_clicks[1, :half], JUNK, C2),
                    _pad_edges(edge_clicks[1, half:], JUNK, C2)])
    zeros_hbm = jnp.zeros((RPT, DP), jnp.float32)

    sc = pl.kernel(
        _sc_body,
        out_type=(jax.ShapeDtypeStruct((NC, NROWS, DP), jnp.float32),
                  jax.ShapeDtypeStruct((NC, NROWS, DP), jnp.float32)),
        mesh=plsc.VectorSubcoreMesh(core_axis_name="c", subcore_axis_name="s"),
        scratch_types=[
            pltpu.VMEM((2, K), jnp.int32),
            pltpu.VMEM((2, K), jnp.int32),
            pltpu.VMEM((2, K, DP), jnp.float32),
            pltpu.VMEM_SHARED((NROWS, DP), jnp.float32),
            pltpu.SemaphoreType.DMA,
            pltpu.SemaphoreType.DMA,
        ],
        compiler_params=pltpu.CompilerParams(use_tc_tiling_on_sc=False),
    )
    out1, out2 = sc(whp_f, whp_c, whp_cb, s1, d1, s2, d2, zeros_hbm)

    grid = N_NODE // BM
    h_user, h_item = pl.pallas_call(
        _comb_body,
        grid=(grid,),
        in_specs=[pl.BlockSpec((NC, BM, DP), lambda i: (0, i, 0)),
                  pl.BlockSpec((NC, BM, DP), lambda i: (0, i, 0))],
        out_specs=(pl.BlockSpec((BM, D), lambda i: (i, 0)),
                   pl.BlockSpec((BM, D), lambda i: (i, 0))),
        out_shape=(jax.ShapeDtypeStruct((N_NODE, D), jnp.float32),
                   jax.ShapeDtypeStruct((N_NODE, D), jnp.float32)),
    )(out1, out2)
    return (h_user, h_item)
